# baseline (speedup 1.0000x reference)
import functools
import math

import jax
import jax.numpy as jnp
from jax import lax
from jax.experimental import pallas as pl
from jax.experimental.pallas import tpu as pltpu
from jax.experimental.pallas import tpu_sc as plsc

_EB = 128
_NWORK = 32
_ROWPAD = 512
_DUMP = 512


def _rup(a, b):
    return (a + b - 1) // b * b


@functools.partial(jax.jit, static_argnames=("out_rows",))
def _sc_segsum(table, gidx, sidx, *, out_rows):
    nb = gidx.shape[0]
    assert nb % _NWORK == 0
    nb_per = nb // _NWORK
    aggr = out_rows + _DUMP
    assert aggr % 128 == 0
    rows_per16 = aggr // 16
    assert rows_per16 % 8 == 0
    orow = out_rows // 16

    mesh = plsc.VectorSubcoreMesh(core_axis_name="c", subcore_axis_name="s")

    @functools.partial(
        pl.kernel,
        mesh=mesh,
        out_type=jax.ShapeDtypeStruct((2, out_rows, 128), jnp.float32),
        scratch_types=[
            pltpu.VMEM_SHARED((aggr, 128), jnp.float32),
            pltpu.VMEM((_EB,), jnp.int32),
            pltpu.VMEM((_EB,), jnp.int32),
            pltpu.VMEM((_EB, 128), jnp.float32),
            pltpu.VMEM((8, 128), jnp.float32),
            pltpu.SemaphoreType.DMA,
        ],
    )
    def k(table_h, gidx_h, sidx_h, out_h, agg_s, gi_v, si_v, rows_v, z_v, sem):
        c = lax.axis_index("c")
        s = lax.axis_index("s")
        wid = c * 16 + s

        zero16 = jnp.zeros((16,), jnp.float32)
        for i in range(8):
            for j in range(8):
                z_v[i, pl.ds(j * 16, 16)] = zero16

        zbase = s * rows_per16

        def zbody(i, carry):
            pltpu.sync_copy(z_v, agg_s.at[pl.ds(zbase + i * 8, 8)])
            return carry

        lax.fori_loop(0, rows_per16 // 8, zbody, 0)
        plsc.subcore_barrier()

        base_b = wid * nb_per

        def body(t, carry):
            j = base_b + t
            pltpu.sync_copy(gidx_h.at[j], gi_v)
            pltpu.sync_copy(sidx_h.at[j], si_v)
            pltpu.async_copy(table_h.at[gi_v], rows_v, sem).wait()
            pltpu.sync_copy(rows_v, agg_s.at[si_v], add=True)
            return carry

        lax.fori_loop(0, nb_per, body, 0)
        plsc.subcore_barrier()

        pltpu.sync_copy(
            agg_s.at[pl.ds(s * orow, orow)],
            out_h.at[c, pl.ds(s * orow, orow)],
        )

    return k(table, gidx, sidx)


def _gelu(h):
    return 0.5 * h * (1.0 + lax.erf(h / math.sqrt(2.0)))


def _mlp_body(n_valid, blk, ngroups, *refs):
    pps = refs[:ngroups]
    w1s = refs[ngroups:2 * ngroups]
    b1, w2, b2, w3, b3, gamma, beta, out = refs[2 * ngroups:]
    h = b1[...].astype(jnp.float32)
    acc = None
    for g in range(ngroups):
        xg = pps[g][0] + pps[g][1]
        part = jnp.dot(xg, w1s[g][...], preferred_element_type=jnp.float32)
        acc = part if acc is None else acc + part
    h = acc + b1[...]
    h = _gelu(h)
    h = jnp.dot(h, w2[...], preferred_element_type=jnp.float32) + b2[...]
    h = _gelu(h)
    h = jnp.dot(h, w3[...], preferred_element_type=jnp.float32) + b3[...]
    mu = jnp.mean(h, axis=-1, keepdims=True)
    var = jnp.mean((h - mu) ** 2, axis=-1, keepdims=True)
    y = (h - mu) / jnp.sqrt(var + 1e-5) * gamma[...] + beta[...]
    rid = pl.program_id(0) * blk + lax.broadcasted_iota(jnp.int32, (blk, 1), 0)
    out[...] = jnp.where(rid < n_valid, y, 0.0)


def _mlp(pps, w1s, b1, w2, b2, w3, b3, gamma, beta, n_valid):
    n_pad = pps[0].shape[1]
    blk = 512
    grid = (n_pad // blk,)
    g = len(pps)
    in_specs = (
        [pl.BlockSpec((2, blk, 128), lambda i: (0, i, 0)) for _ in range(g)]
        + [pl.BlockSpec((128, 128), lambda i: (0, 0)) for _ in range(g)]
        + [pl.BlockSpec((1, 128), lambda i: (0, 0)),
           pl.BlockSpec((128, 128), lambda i: (0, 0)),
           pl.BlockSpec((1, 128), lambda i: (0, 0)),
           pl.BlockSpec((128, 128), lambda i: (0, 0)),
           pl.BlockSpec((1, 128), lambda i: (0, 0)),
           pl.BlockSpec((1, 128), lambda i: (0, 0)),
           pl.BlockSpec((1, 128), lambda i: (0, 0))]
    )
    return pl.pallas_call(
        functools.partial(_mlp_body, n_valid, blk, g),
        grid=grid,
        in_specs=in_specs,
        out_specs=pl.BlockSpec((blk, 128), lambda i: (i, 0)),
        out_shape=jax.ShapeDtypeStruct((n_pad, 128), jnp.float32),
    )(*pps, *w1s, b1, w2, b2, w3, b3, gamma, beta)


def _scale_body(x_ref, s_ref, o_ref):
    o_ref[...] = x_ref[...] * s_ref[...]


def _scale_rows(x_pad, s_bcast):
    n_pad = x_pad.shape[0]
    blk = 512
    return pl.pallas_call(
        _scale_body,
        grid=(n_pad // blk,),
        in_specs=[pl.BlockSpec((blk, 128), lambda i: (i, 0)),
                  pl.BlockSpec((blk, 128), lambda i: (i, 0))],
        out_specs=pl.BlockSpec((blk, 128), lambda i: (i, 0)),
        out_shape=jax.ShapeDtypeStruct((n_pad, 128), jnp.float32),
    )(x_pad, s_bcast)


def _prep_params(p):
    w1, b1, w2, b2, w3, b3, gamma, beta = p
    w1t = w1.T
    return (w1t, b1.reshape(1, -1), w2.T, b2.reshape(1, -1), w3.T,
            b3.reshape(1, -1), gamma.reshape(1, -1), beta.reshape(1, -1))


def _edge_blocks(gidx, sidx, zrow, dump_base):
    m = gidx.shape[0]
    cap = _rup(m, _NWORK * _EB)
    pad = cap - m
    gidx = jnp.concatenate([gidx, jnp.full((pad,), zrow, jnp.int32)])
    sidx = jnp.concatenate(
        [sidx, dump_base + (jnp.arange(pad, dtype=jnp.int32) % _DUMP)])
    return gidx.reshape(cap // _EB, _EB), sidx.reshape(cap // _EB, _EB)


def kernel(x, edge_index, pool_ws, down_params, up_params):
    n0, cdim = x.shape
    e = edge_index.shape[1]
    depth = len(pool_ws)
    senders = edge_index[0]
    receivers = edge_index[1]

    n_pad0 = _rup(n0, _ROWPAD)
    x_pad = jnp.pad(x, ((0, n_pad0 - n0), (0, 0)))

    cur_x = x_pad
    cur_n = n0
    s_cur, r_cur = senders, receivers
    valid_cur = jnp.ones((e,), jnp.bool_)

    xs_pad = [x_pad]
    ns = [n0]
    edges_lvl = [(senders, receivers, valid_cur)]
    newidx_lvl = []

    for i in range(depth):
        w = pool_ws[i]
        score = jnp.tanh((cur_x[:cur_n] @ w) / jnp.linalg.norm(w))
        k = int(math.ceil(0.5 * cur_n))
        vals, perm = lax.top_k(score, k)
        k_pad = _rup(k, _ROWPAD)
        new_idx = jnp.full((cur_n,), -1, jnp.int32).at[perm].set(
            jnp.arange(k, dtype=jnp.int32))

        scale = jnp.zeros((_rup(cur_n, _ROWPAD),), jnp.float32).at[perm].set(vals)
        table = _scale_rows(cur_x, jnp.broadcast_to(scale[:, None],
                                                    (cur_x.shape[0], 128)))

        s_new = jnp.take(new_idx, s_cur)
        r_new = jnp.take(new_idx, r_cur)
        valid_new = (s_new >= 0) & (r_new >= 0) & valid_cur

        v2 = jnp.concatenate([valid_new, valid_new])
        g_old = jnp.concatenate([s_cur, r_cur])
        sc_new = jnp.concatenate([r_new, s_new])
        zrow = jnp.int32(cur_n)
        gidx = jnp.where(v2, g_old, zrow)
        dump = k_pad + (jnp.arange(2 * e, dtype=jnp.int32) % _DUMP)
        sidx = jnp.where(v2, sc_new, dump)
        gb, sb = _edge_blocks(gidx, sidx, cur_n, k_pad)
        parts = _sc_segsum(table, gb, sb, out_rows=k_pad)

        dp = _prep_params(down_params[i])
        cur_x = _mlp([parts], [dp[0]], *dp[1:], n_valid=k)

        s_store = jnp.where(valid_new, s_new, 0)
        r_store = jnp.where(valid_new, r_new, 0)
        newidx_lvl.append(new_idx)
        cur_n = k
        s_cur, r_cur, valid_cur = s_store, r_store, valid_new
        if i < depth - 1:
            xs_pad.append(cur_x)
            ns.append(k)
            edges_lvl.append((s_store, r_store, valid_new))

    for i in range(depth):
        j = depth - 1 - i
        res = xs_pad[j]
        n_j = ns[j]
        n_j_pad = res.shape[0]
        s_j, r_j, valid_j = edges_lvl[j]
        inv = newidx_lvl[j]
        child_n = cur_n
        child_pad = cur_x.shape[0]

        v2 = jnp.concatenate([valid_j, valid_j])
        g_res = jnp.concatenate([s_j, r_j])
        sc_j = jnp.concatenate([r_j, s_j])
        dump = n_j_pad + (jnp.arange(2 * e, dtype=jnp.int32) % _DUMP)
        gidx_res = jnp.where(v2, g_res, jnp.int32(n_j))
        sidx = jnp.where(v2, sc_j, dump)

        up_ids = jnp.take(inv, g_res)
        vu = v2 & (up_ids >= 0)
        gidx_up = jnp.where(vu, up_ids, jnp.int32(child_n))

        gb_r, sb = _edge_blocks(gidx_res, sidx, n_j, n_j_pad)
        gb_u, _ = _edge_blocks(gidx_up, sidx, child_n, n_j_pad)

        parts_res = _sc_segsum(res, gb_r, sb, out_rows=n_j_pad)
        parts_up = _sc_segsum(cur_x, gb_u, sb, out_rows=n_j_pad)

        up = _prep_params(up_params[i])
        w1a = up[0][:cdim]
        w1b = up[0][cdim:]
        cur_x = _mlp([parts_res, parts_up], [w1a, w1b], *up[1:], n_valid=n_j)
        cur_n = n_j

    return cur_x[:n0]
